# baseline (speedup 1.0000x reference)
import jax
import jax.numpy as jnp
from jax import lax
from jax.experimental import pallas as pl
from jax.experimental.pallas import tpu as pltpu
from jax.experimental.pallas import tpu_sc as plsc

B = 128
V = 16
D = 256
L = 16
NC = 2
NS = 16
NW = NC * NS
BPW = B // NW
NCHUNK = 4
CROWS = (V * V) // NCHUNK
IPC = V // NCHUNK


def _fill_chunk(hv_v, hs, buf, slot, g, write_right):
    for il in range(IPC):
        i = g * IPC + il
        lv = [hv_v[hs, i, pl.ds(c * L, L)] for c in range(D // L)]

        @plsc.parallel_loop(0, V, 1, unroll=4)
        def _rows(r):
            row = il * V + r
            for c in range(D // L):
                buf[slot, row, pl.ds(c * L, L)] = lv[c]
            if write_right:
                for c in range(D // L):
                    buf[slot, row, pl.ds(D + c * L, L)] = hv_v[hs, r, pl.ds(c * L, L)]


def _node_to_edge_body(hv_hbm, out_hbm, hv_v, buf, sem_hv, sem0, sem1):
    wid = lax.axis_index("s") * NC + lax.axis_index("c")
    b0 = wid * BPW
    sems = (sem0, sem1)

    pltpu.sync_copy(hv_hbm.at[b0], hv_v.at[0])

    def batch_body(bi, _):
        b = b0 + bi
        hs = bi & 1
        if True:
            nxt = jnp.where(bi + 1 < BPW, b + 1, b0)
            hv_pf = pltpu.async_copy(hv_hbm.at[nxt], hv_v.at[1 - hs], sem_hv)

        copies = [None, None]
        for g in range(NCHUNK):
            slot = g % 2
            if copies[slot] is not None:
                copies[slot].wait()
            _fill_chunk(hv_v, hs, buf, slot, g, write_right=(g < 2))
            copies[slot] = pltpu.async_copy(
                buf.at[slot], out_hbm.at[b, pl.ds(g * CROWS, CROWS)], sems[slot]
            )
        copies[0].wait()
        copies[1].wait()
        hv_pf.wait()
        return 0

    lax.fori_loop(0, BPW, batch_body, 0, unroll=False)


@jax.jit
def kernel(hv):
    mesh = plsc.VectorSubcoreMesh(core_axis_name="c", subcore_axis_name="s")
    out = pl.kernel(
        _node_to_edge_body,
        out_type=jax.ShapeDtypeStruct((B, V * V, 2 * D), jnp.float32),
        mesh=mesh,
        scratch_types=[
            pltpu.VMEM((2, V, D), jnp.float32),
            pltpu.VMEM((2, CROWS, 2 * D), jnp.float32),
            pltpu.SemaphoreType.DMA,
            pltpu.SemaphoreType.DMA,
            pltpu.SemaphoreType.DMA,
        ],
    )(hv)
    return out.reshape(B, V, V, 2 * D)


# speedup vs baseline: 1.3112x; 1.3112x over previous
import jax
import jax.numpy as jnp
from jax import lax
from jax.experimental import pallas as pl
from jax.experimental.pallas import tpu as pltpu
from jax.experimental.pallas import tpu_sc as plsc

B = 128
V = 16
D = 256
L = 16
NC = 2
NS = 16
NW = NC * NS
BPW = B // NW
NCHUNK = 2
CROWS = (V * V) // NCHUNK
IPC = V // NCHUNK


def _fill_left(hv_v, hs, buf, slot, g):
    for il in range(IPC):
        i = g * IPC + il
        lv = [hv_v[hs, i, pl.ds(c * L, L)] for c in range(D // L)]

        @plsc.parallel_loop(0, V, 1, unroll=4)
        def _rows(r):
            row = il * V + r
            for c in range(D // L):
                buf[slot, row, pl.ds(c * L, L)] = lv[c]


def _node_to_edge_body(hv_hbm, out_hbm, hv_v, buf, sem_hv, sem_r, sem0, sem1):
    wid = lax.axis_index("s") * NC + lax.axis_index("c")
    b0 = wid * BPW
    sems = (sem0, sem1)

    pltpu.sync_copy(hv_hbm.at[b0], hv_v.at[0])

    def batch_body(bi, _):
        b = b0 + bi
        hs = bi & 1
        nxt = jnp.where(bi + 1 < BPW, b + 1, b0)
        hv_pf = pltpu.async_copy(hv_hbm.at[nxt], hv_v.at[1 - hs], sem_hv)

        right = [
            pltpu.async_copy(
                hv_v.at[hs],
                out_hbm.at[b, pl.ds(i * V, V), pl.ds(D, D)],
                sem_r,
            )
            for i in range(V)
        ]

        copies = [None, None]
        for g in range(NCHUNK):
            slot = g % 2
            if copies[slot] is not None:
                copies[slot].wait()
            _fill_left(hv_v, hs, buf, slot, g)
            copies[slot] = pltpu.async_copy(
                buf.at[slot],
                out_hbm.at[b, pl.ds(g * CROWS, CROWS), pl.ds(0, D)],
                sems[slot],
            )
        copies[0].wait()
        copies[1].wait()
        for cp in right:
            cp.wait()
        hv_pf.wait()
        return 0

    lax.fori_loop(0, BPW, batch_body, 0, unroll=False)


@jax.jit
def kernel(hv):
    mesh = plsc.VectorSubcoreMesh(core_axis_name="c", subcore_axis_name="s")
    out = pl.kernel(
        _node_to_edge_body,
        out_type=jax.ShapeDtypeStruct((B, V * V, 2 * D), jnp.float32),
        mesh=mesh,
        scratch_types=[
            pltpu.VMEM((2, V, D), jnp.float32),
            pltpu.VMEM((2, CROWS, D), jnp.float32),
            pltpu.SemaphoreType.DMA,
            pltpu.SemaphoreType.DMA,
            pltpu.SemaphoreType.DMA,
            pltpu.SemaphoreType.DMA,
        ],
    )(hv)
    return out.reshape(B, V, V, 2 * D)


# speedup vs baseline: 1.4497x; 1.1056x over previous
import jax
import jax.numpy as jnp
from jax import lax
from jax.experimental import pallas as pl
from jax.experimental.pallas import tpu as pltpu
from jax.experimental.pallas import tpu_sc as plsc

B = 128
V = 16
D = 256
NC = 2
NS = 16
NW = NC * NS
BPW = B // NW


def _node_to_edge_body(hv_hbm, out_hbm, hv_v, sem_hv, sem_out):
    wid = lax.axis_index("s") * NC + lax.axis_index("c")
    b0 = wid * BPW

    hv_loads = [
        pltpu.async_copy(hv_hbm.at[b0 + k], hv_v.at[k], sem_hv)
        for k in range(BPW)
    ]
    for bi in range(BPW):
        b = b0 + bi
        hv_loads[bi].wait()
        copies = []
        for i in range(V):
            copies.append(
                pltpu.async_copy(
                    hv_v.at[bi], out_hbm.at[b, i, :, pl.ds(D, D)], sem_out
                )
            )
            copies.append(
                pltpu.async_copy(
                    hv_v.at[bi], out_hbm.at[b, :, i, pl.ds(0, D)], sem_out
                )
            )
        for cp in copies:
            cp.wait()


@jax.jit
def kernel(hv):
    mesh = plsc.VectorSubcoreMesh(core_axis_name="c", subcore_axis_name="s")
    out = pl.kernel(
        _node_to_edge_body,
        out_type=jax.ShapeDtypeStruct((B, V, V, 2 * D), jnp.float32),
        mesh=mesh,
        scratch_types=[
            pltpu.VMEM((BPW, V, D), jnp.float32),
            pltpu.SemaphoreType.DMA,
            pltpu.SemaphoreType.DMA,
        ],
    )(hv)
    return out
